# baseline (speedup 1.0000x reference)
import functools

import jax
import jax.numpy as jnp
from jax import lax
from jax.experimental import pallas as pl
from jax.experimental.pallas import tpu as pltpu
from jax.experimental.pallas import tpu_sc as plsc

N = 10000
E = 640000
B = 64
K = 5
K2 = K * K

NTILES = 32
CH = 32
NCHUNK = 20000
CPT = NCHUNK // NTILES
EP = NCHUNK * CH
NP = 10240
RPT = NP // 16


_PCB = 32


def _prep_body(s_ref, d_ref, p0_ref, p1_ref, ei_ref, ew_ref):
    p0 = p0_ref[...]
    p1 = p1_ref[...]
    v0 = jnp.clip(p0, 0.0, 1.0) * (K - 1.0)
    v1 = jnp.clip(p1, 0.0, 1.0) * (K - 1.0)
    i00f = jnp.clip(jnp.floor(v0), 0.0, K - 1.0)
    i10f = jnp.clip(jnp.floor(v1), 0.0, K - 1.0)
    f0 = v0 - i00f
    f1 = v1 - i10f
    i00 = i00f.astype(jnp.int32)
    i01 = jnp.minimum(i00 + 1, K - 1)
    i10 = i10f.astype(jnp.int32)
    i11 = jnp.minimum(i10 + 1, K - 1)
    row = lax.broadcasted_iota(jnp.int32, (_PCB, CH), 0) + pl.program_id(0) * _PCB
    col = lax.broadcasted_iota(jnp.int32, (_PCB, CH), 1)
    valid = ((row * CH + col) < E).astype(jnp.float32)
    base = s_ref[...] * K2
    ei_ref[:, 0, :] = base + i00 * K + i10
    ei_ref[:, 1, :] = base + i00 * K + i11
    ei_ref[:, 2, :] = base + i01 * K + i10
    ei_ref[:, 3, :] = base + i01 * K + i11
    ei_ref[:, 4, :] = d_ref[...]
    ew_ref[:, 0, :] = (1.0 - f0) * (1.0 - f1) * valid
    ew_ref[:, 1, :] = (1.0 - f0) * f1 * valid
    ew_ref[:, 2, :] = f0 * (1.0 - f1) * valid
    ew_ref[:, 3, :] = f0 * f1 * valid


def _prep(src2, dst2, p02, p12):
    return pl.pallas_call(
        _prep_body,
        grid=(NCHUNK // _PCB,),
        in_specs=[pl.BlockSpec((_PCB, CH), lambda i: (i, 0))] * 4,
        out_specs=(pl.BlockSpec((_PCB, 5, CH), lambda i: (i, 0, 0)),
                   pl.BlockSpec((_PCB, 4, CH), lambda i: (i, 0, 0))),
        out_shape=(jax.ShapeDtypeStruct((NCHUNK, 5, CH), jnp.int32),
                   jax.ShapeDtypeStruct((NCHUNK, 4, CH), jnp.float32)),
    )(src2, dst2, p02, p12)


def _table_body(x_ref, w_ref, o_ref):
    o_ref[...] = jnp.dot(x_ref[...], w_ref[...],
                         preferred_element_type=jnp.float32)


def _make_table(x, Wf):
    cin = x.shape[1]
    w = Wf.shape[1]
    Nb = 2000
    return pl.pallas_call(
        _table_body,
        grid=(N // Nb,),
        in_specs=[pl.BlockSpec((Nb, cin), lambda i: (i, 0)),
                  pl.BlockSpec((cin, w), lambda i: (0, 0))],
        out_specs=pl.BlockSpec((Nb, w), lambda i: (i, 0)),
        out_shape=jax.ShapeDtypeStruct((N, w), jnp.float32),
    )(x, Wf)


def _sc_agg(table2d, eidx, ew, cp):
    grp = cp // 16
    mesh = plsc.VectorSubcoreMesh(core_axis_name="c", subcore_axis_name="s")

    @functools.partial(
        pl.kernel,
        out_type=jax.ShapeDtypeStruct((2, NP, cp), jnp.float32),
        mesh=mesh,
        compiler_params=pltpu.CompilerParams(use_tc_tiling_on_sc=False),
        scratch_types=[
            pltpu.VMEM((2, 5, CH), jnp.int32),
            pltpu.VMEM((2, 4, CH), jnp.float32),
            pltpu.VMEM((2, 4, CH, cp), jnp.float32),
            pltpu.VMEM((CH, cp), jnp.float32),
            pltpu.VMEM_SHARED((NP, cp), jnp.float32),
            pltpu.SemaphoreType.DMA,
            pltpu.SemaphoreType.DMA,
        ],
    )
    def sc_kernel(table_hbm, eidx_hbm, ew_hbm, out_hbm,
                  exv, ewv, rb, yb, acc, sema, semb):
        c = lax.axis_index("c")
        s = lax.axis_index("s")
        gw = c * 16 + s

        def zrow(i, carry):
            for j in range(grp):
                yb[i, pl.ds(j * 16, 16)] = jnp.zeros((16,), jnp.float32)
            return carry
        lax.fori_loop(0, CH, zrow, 0)
        base = s * RPT
        for i in range(RPT // CH):
            pltpu.sync_copy(yb, acc.at[pl.ds(base + i * CH, CH)])
        plsc.subcore_barrier()

        def stage(t, pb, sem):
            ci = gw * CPT + t
            pltpu.sync_copy(eidx_hbm.at[ci], exv.at[pb])
            pltpu.sync_copy(ew_hbm.at[ci], ewv.at[pb])
            for j in range(4):
                pltpu.async_copy(table_hbm.at[exv.at[pb, j]], rb.at[pb, j],
                                 sem)

        def wait_gathers(pb, sem):
            for j in range(4):
                pltpu.make_async_copy(table_hbm.at[exv.at[pb, j]],
                                      rb.at[pb, j], sem).wait()

        def chunk_body(t, p):
            @pl.when((t + 1 < CPT) & (p == 0))
            def _s1():
                stage(t + 1, 1, semb)

            @pl.when((t + 1 < CPT) & (p == 1))
            def _s0():
                stage(t + 1, 0, sema)

            @pl.when(p == 0)
            def _w0():
                wait_gathers(0, sema)

            @pl.when(p == 1)
            def _w1():
                wait_gathers(1, semb)

            def ebody(g, ecarry):
                gsl = pl.ds(g * 16, 16)
                wv0 = ewv[p, 0, gsl]
                wv1 = ewv[p, 1, gsl]
                wv2 = ewv[p, 2, gsl]
                wv3 = ewv[p, 3, gsl]
                for l in range(16):
                    e = g * 16 + l
                    w0 = wv0[l]
                    w1 = wv1[l]
                    w2 = wv2[l]
                    w3 = wv3[l]
                    for j in range(grp):
                        sl = pl.ds(j * 16, 16)
                        yb[e, sl] = (w0 * rb[p, 0, e, sl] + w1 * rb[p, 1, e, sl]
                                     + w2 * rb[p, 2, e, sl] + w3 * rb[p, 3, e, sl])
                return ecarry
            lax.fori_loop(0, CH // 16, ebody, 0)

            @pl.when(p == 0)
            def _a0():
                pltpu.sync_copy(yb, acc.at[exv.at[0, 4]], add=True)

            @pl.when(p == 1)
            def _a1():
                pltpu.sync_copy(yb, acc.at[exv.at[1, 4]], add=True)

            return 1 - p
        stage(0, 0, sema)
        lax.fori_loop(0, CPT, chunk_body, jnp.int32(0))

        plsc.subcore_barrier()
        pltpu.sync_copy(acc.at[pl.ds(base, RPT)],
                        out_hbm.at[c, pl.ds(base, RPT)])

    return sc_kernel(table2d, eidx, ew)


def _elu(h):
    return jnp.where(h > 0, h, jnp.exp(h) - 1.0)


def _finish1_body(p_ref, x_ref, r_ref, b_ref, h_ref, d_ref):
    a = p_ref[0] + p_ref[1]
    deg = jnp.maximum(a[:, 32:33], 1.0)
    h = a[:, :32] / deg + jnp.dot(x_ref[...], r_ref[...],
                                  preferred_element_type=jnp.float32) + b_ref[...]
    h_ref[...] = _elu(h)
    d_ref[...] = deg


def _finish1(part, x, root, bias):
    return pl.pallas_call(
        _finish1_body,
        out_shape=(jax.ShapeDtypeStruct((N, 32), jnp.float32),
                   jax.ShapeDtypeStruct((N, 1), jnp.float32)),
    )(part, x, root, bias)


def _finish_body(cout, p_ref, x_ref, r_ref, b_ref, d_ref, h_ref):
    a = p_ref[0] + p_ref[1]
    agg = a[:, :cout] if cout != a.shape[1] else a
    h = agg / d_ref[...] + jnp.dot(x_ref[...], r_ref[...],
                                   preferred_element_type=jnp.float32) + b_ref[...]
    h_ref[...] = _elu(h)


def _finish(part, x, root, bias, deg, cout):
    return pl.pallas_call(
        functools.partial(_finish_body, cout),
        out_shape=jax.ShapeDtypeStruct((N, cout), jnp.float32),
    )(part, x, root, bias, deg)


def _pool_fc_body(sl_ref, h_ref, wfc_ref, bfc_ref, out_ref):
    n_iota1 = lax.broadcasted_iota(jnp.int32, (1, N), 1)
    seg = jnp.zeros((1, N), jnp.int32)
    for j in range(1, B):
        seg = seg + jnp.where(sl_ref[j] <= n_iota1, 1, 0)
    b_iota = lax.broadcasted_iota(jnp.int32, (B, N), 0)
    M = (seg == b_iota).astype(jnp.float32)
    cnt = jnp.sum(M, axis=1, keepdims=True)
    sums = jnp.dot(M, h_ref[...], preferred_element_type=jnp.float32)
    g = sums / jnp.maximum(cnt, 1.0)
    logits = jnp.dot(g, wfc_ref[...], preferred_element_type=jnp.float32) + bfc_ref[...]
    m = jnp.max(logits, axis=1, keepdims=True)
    z = logits - m
    out_ref[...] = z - jnp.log(jnp.sum(jnp.exp(z), axis=1, keepdims=True))


def _pool_fc(h, sl, Wfc, bfc):
    return pl.pallas_call(
        _pool_fc_body,
        out_shape=jax.ShapeDtypeStruct((B, Wfc.shape[1]), jnp.float32),
        in_specs=[
            pl.BlockSpec(memory_space=pltpu.SMEM),
            pl.BlockSpec(memory_space=pltpu.VMEM),
            pl.BlockSpec(memory_space=pltpu.VMEM),
            pl.BlockSpec(memory_space=pltpu.VMEM),
        ],
        out_specs=pl.BlockSpec(memory_space=pltpu.VMEM),
    )(sl, h, Wfc, bfc)


def kernel(x, edge_index, pseudo, slice_idx, W1, root1, b1, W2, root2, b2,
           W3, root3, b3, Wfc, bfc):
    pad = EP - E
    src2 = jnp.pad(edge_index[0], (0, pad)).reshape(NCHUNK, CH)
    dst2 = jnp.pad(edge_index[1], (0, pad)).reshape(NCHUNK, CH)
    p02 = jnp.pad(pseudo[:, 0], (0, pad)).reshape(NCHUNK, CH)
    p12 = jnp.pad(pseudo[:, 1], (0, pad)).reshape(NCHUNK, CH)
    eidx, ew = _prep(src2, dst2, p02, p12)

    x1 = jnp.concatenate([x, jnp.ones((N, 1), x.dtype)], axis=1)
    W1p = (jnp.zeros((K2, 9, 48), jnp.float32)
           .at[:, :8, :32].set(W1)
           .at[:, 8, 32].set(1.0))
    t1 = _make_table(x1, W1p.transpose(1, 0, 2).reshape(9, K2 * 48))
    part1 = _sc_agg(t1.reshape(N * K2, 48), eidx, ew, 48)
    h1, deg = _finish1(part1[:, :N, :], x, root1, b1)

    t2 = _make_table(h1, W2.transpose(1, 0, 2).reshape(32, K2 * 64))
    part2 = _sc_agg(t2.reshape(N * K2, 64), eidx, ew, 64)
    h2 = _finish(part2[:, :N, :], h1, root2, b2, deg, 64)

    W3p = jnp.pad(W3, ((0, 0), (0, 0), (0, 4)))
    t3 = _make_table(h2, W3p.transpose(1, 0, 2).reshape(64, K2 * 128))
    part3 = _sc_agg(t3.reshape(N * K2, 128), eidx, ew, 128)
    h3 = _finish(part3[:, :N, :], h2, root3, b3, deg, 124)

    return _pool_fc(h3, slice_idx, Wfc, bfc)


# speedup vs baseline: 1.2687x; 1.2687x over previous
import functools

import jax
import jax.numpy as jnp
from jax import lax
from jax.experimental import pallas as pl
from jax.experimental.pallas import tpu as pltpu
from jax.experimental.pallas import tpu_sc as plsc

N = 10000
E = 640000
B = 64
K = 5
K2 = K * K

NTILES = 32
CH = 64
NCHUNK = 10016
CPT = NCHUNK // NTILES
EP = NCHUNK * CH
NP = 10240
RPT = NP // 16


_PCB = 32


def _prep_body(s_ref, d_ref, p0_ref, p1_ref, ei_ref, ew_ref):
    p0 = p0_ref[...]
    p1 = p1_ref[...]
    v0 = jnp.clip(p0, 0.0, 1.0) * (K - 1.0)
    v1 = jnp.clip(p1, 0.0, 1.0) * (K - 1.0)
    i00f = jnp.clip(jnp.floor(v0), 0.0, K - 1.0)
    i10f = jnp.clip(jnp.floor(v1), 0.0, K - 1.0)
    f0 = v0 - i00f
    f1 = v1 - i10f
    i00 = i00f.astype(jnp.int32)
    i01 = jnp.minimum(i00 + 1, K - 1)
    i10 = i10f.astype(jnp.int32)
    i11 = jnp.minimum(i10 + 1, K - 1)
    row = lax.broadcasted_iota(jnp.int32, (_PCB, CH), 0) + pl.program_id(0) * _PCB
    col = lax.broadcasted_iota(jnp.int32, (_PCB, CH), 1)
    valid = ((row * CH + col) < E).astype(jnp.float32)
    base = s_ref[...] * K2
    ei_ref[:, 0, :] = base + i00 * K + i10
    ei_ref[:, 1, :] = base + i00 * K + i11
    ei_ref[:, 2, :] = base + i01 * K + i10
    ei_ref[:, 3, :] = base + i01 * K + i11
    ei_ref[:, 4, :] = d_ref[...]
    ew_ref[:, 0, :] = (1.0 - f0) * (1.0 - f1) * valid
    ew_ref[:, 1, :] = (1.0 - f0) * f1 * valid
    ew_ref[:, 2, :] = f0 * (1.0 - f1) * valid
    ew_ref[:, 3, :] = f0 * f1 * valid


def _prep(src2, dst2, p02, p12):
    return pl.pallas_call(
        _prep_body,
        grid=(NCHUNK // _PCB,),
        in_specs=[pl.BlockSpec((_PCB, CH), lambda i: (i, 0))] * 4,
        out_specs=(pl.BlockSpec((_PCB, 5, CH), lambda i: (i, 0, 0)),
                   pl.BlockSpec((_PCB, 4, CH), lambda i: (i, 0, 0))),
        out_shape=(jax.ShapeDtypeStruct((NCHUNK, 5, CH), jnp.int32),
                   jax.ShapeDtypeStruct((NCHUNK, 4, CH), jnp.float32)),
    )(src2, dst2, p02, p12)


def _table_body(x_ref, w_ref, o_ref):
    o_ref[...] = jnp.dot(x_ref[...], w_ref[...],
                         preferred_element_type=jnp.float32)


def _make_table(x, Wf):
    cin = x.shape[1]
    w = Wf.shape[1]
    Nb = 2000
    return pl.pallas_call(
        _table_body,
        grid=(N // Nb,),
        in_specs=[pl.BlockSpec((Nb, cin), lambda i: (i, 0)),
                  pl.BlockSpec((cin, w), lambda i: (0, 0))],
        out_specs=pl.BlockSpec((Nb, w), lambda i: (i, 0)),
        out_shape=jax.ShapeDtypeStruct((N, w), jnp.float32),
    )(x, Wf)


def _sc_agg(table2d, eidx, ew, cp):
    grp = cp // 16
    mesh = plsc.VectorSubcoreMesh(core_axis_name="c", subcore_axis_name="s")

    @functools.partial(
        pl.kernel,
        out_type=jax.ShapeDtypeStruct((2, NP, cp), jnp.float32),
        mesh=mesh,
        compiler_params=pltpu.CompilerParams(use_tc_tiling_on_sc=False),
        scratch_types=[
            pltpu.VMEM((5, CH), jnp.int32),
            pltpu.VMEM((4, CH), jnp.float32),
            pltpu.VMEM((4, CH, cp), jnp.float32),
            pltpu.VMEM((CH, cp), jnp.float32),
            pltpu.VMEM_SHARED((NP, cp), jnp.float32),
            pltpu.SemaphoreType.DMA,
        ],
    )
    def sc_kernel(table_hbm, eidx_hbm, ew_hbm, out_hbm,
                  exv, ewv, rb, yb, acc, gsem):
        c = lax.axis_index("c")
        s = lax.axis_index("s")
        gw = c * 16 + s

        def zrow(i, carry):
            for j in range(grp):
                yb[i, pl.ds(j * 16, 16)] = jnp.zeros((16,), jnp.float32)
            return carry
        lax.fori_loop(0, CH, zrow, 0)
        base = s * RPT
        for i in range(RPT // CH):
            pltpu.sync_copy(yb, acc.at[pl.ds(base + i * CH, CH)])
        plsc.subcore_barrier()

        def chunk_body(t, carry):
            ci = gw * CPT + t
            pltpu.sync_copy(eidx_hbm.at[ci], exv)
            pltpu.sync_copy(ew_hbm.at[ci], ewv)
            cps = [pltpu.async_copy(table_hbm.at[exv.at[j]], rb.at[j], gsem)
                   for j in range(4)]
            for d in cps:
                d.wait()

            def ebody(g, ecarry):
                gsl = pl.ds(g * 16, 16)
                wv0 = ewv[0, gsl]
                wv1 = ewv[1, gsl]
                wv2 = ewv[2, gsl]
                wv3 = ewv[3, gsl]
                for l in range(16):
                    e = g * 16 + l
                    w0 = wv0[l]
                    w1 = wv1[l]
                    w2 = wv2[l]
                    w3 = wv3[l]
                    for j in range(grp):
                        sl = pl.ds(j * 16, 16)
                        yb[e, sl] = (w0 * rb[0, e, sl] + w1 * rb[1, e, sl]
                                     + w2 * rb[2, e, sl] + w3 * rb[3, e, sl])
                return ecarry
            lax.fori_loop(0, CH // 16, ebody, 0)
            pltpu.sync_copy(yb, acc.at[exv.at[4]], add=True)
            return carry
        lax.fori_loop(0, CPT, chunk_body, 0)

        plsc.subcore_barrier()
        pltpu.sync_copy(acc.at[pl.ds(base, RPT)],
                        out_hbm.at[c, pl.ds(base, RPT)])

    return sc_kernel(table2d, eidx, ew)


def _elu(h):
    return jnp.where(h > 0, h, jnp.exp(h) - 1.0)


def _finish1_body(p_ref, x_ref, r_ref, b_ref, h_ref, d_ref):
    a = p_ref[0] + p_ref[1]
    deg = jnp.maximum(a[:, 32:33], 1.0)
    h = a[:, :32] / deg + jnp.dot(x_ref[...], r_ref[...],
                                  preferred_element_type=jnp.float32) + b_ref[...]
    h_ref[...] = _elu(h)
    d_ref[...] = deg


def _finish1(part, x, root, bias):
    return pl.pallas_call(
        _finish1_body,
        out_shape=(jax.ShapeDtypeStruct((N, 32), jnp.float32),
                   jax.ShapeDtypeStruct((N, 1), jnp.float32)),
    )(part, x, root, bias)


def _finish_body(cout, p_ref, x_ref, r_ref, b_ref, d_ref, h_ref):
    a = p_ref[0] + p_ref[1]
    agg = a[:, :cout] if cout != a.shape[1] else a
    h = agg / d_ref[...] + jnp.dot(x_ref[...], r_ref[...],
                                   preferred_element_type=jnp.float32) + b_ref[...]
    h_ref[...] = _elu(h)


def _finish(part, x, root, bias, deg, cout):
    return pl.pallas_call(
        functools.partial(_finish_body, cout),
        out_shape=jax.ShapeDtypeStruct((N, cout), jnp.float32),
    )(part, x, root, bias, deg)


def _pool_fc_body(sl_ref, h_ref, wfc_ref, bfc_ref, out_ref):
    n_iota1 = lax.broadcasted_iota(jnp.int32, (1, N), 1)
    seg = jnp.zeros((1, N), jnp.int32)
    for j in range(1, B):
        seg = seg + jnp.where(sl_ref[j] <= n_iota1, 1, 0)
    b_iota = lax.broadcasted_iota(jnp.int32, (B, N), 0)
    M = (seg == b_iota).astype(jnp.float32)
    cnt = jnp.sum(M, axis=1, keepdims=True)
    sums = jnp.dot(M, h_ref[...], preferred_element_type=jnp.float32)
    g = sums / jnp.maximum(cnt, 1.0)
    logits = jnp.dot(g, wfc_ref[...], preferred_element_type=jnp.float32) + bfc_ref[...]
    m = jnp.max(logits, axis=1, keepdims=True)
    z = logits - m
    out_ref[...] = z - jnp.log(jnp.sum(jnp.exp(z), axis=1, keepdims=True))


def _pool_fc(h, sl, Wfc, bfc):
    return pl.pallas_call(
        _pool_fc_body,
        out_shape=jax.ShapeDtypeStruct((B, Wfc.shape[1]), jnp.float32),
        in_specs=[
            pl.BlockSpec(memory_space=pltpu.SMEM),
            pl.BlockSpec(memory_space=pltpu.VMEM),
            pl.BlockSpec(memory_space=pltpu.VMEM),
            pl.BlockSpec(memory_space=pltpu.VMEM),
        ],
        out_specs=pl.BlockSpec(memory_space=pltpu.VMEM),
    )(sl, h, Wfc, bfc)


def kernel(x, edge_index, pseudo, slice_idx, W1, root1, b1, W2, root2, b2,
           W3, root3, b3, Wfc, bfc):
    pad = EP - E
    src2 = jnp.pad(edge_index[0], (0, pad)).reshape(NCHUNK, CH)
    dst2 = jnp.pad(edge_index[1], (0, pad)).reshape(NCHUNK, CH)
    p02 = jnp.pad(pseudo[:, 0], (0, pad)).reshape(NCHUNK, CH)
    p12 = jnp.pad(pseudo[:, 1], (0, pad)).reshape(NCHUNK, CH)
    eidx, ew = _prep(src2, dst2, p02, p12)

    x1 = jnp.concatenate([x, jnp.ones((N, 1), x.dtype)], axis=1)
    W1p = (jnp.zeros((K2, 9, 48), jnp.float32)
           .at[:, :8, :32].set(W1)
           .at[:, 8, 32].set(1.0))
    t1 = _make_table(x1, W1p.transpose(1, 0, 2).reshape(9, K2 * 48))
    part1 = _sc_agg(t1.reshape(N * K2, 48), eidx, ew, 48)
    h1, deg = _finish1(part1[:, :N, :], x, root1, b1)

    t2 = _make_table(h1, W2.transpose(1, 0, 2).reshape(32, K2 * 64))
    part2 = _sc_agg(t2.reshape(N * K2, 64), eidx, ew, 64)
    h2 = _finish(part2[:, :N, :], h1, root2, b2, deg, 64)

    W3p = jnp.pad(W3, ((0, 0), (0, 0), (0, 4)))
    t3 = _make_table(h2, W3p.transpose(1, 0, 2).reshape(64, K2 * 128))
    part3 = _sc_agg(t3.reshape(N * K2, 128), eidx, ew, 128)
    h3 = _finish(part3[:, :N, :], h2, root3, b3, deg, 124)

    return _pool_fc(h3, slice_idx, Wfc, bfc)


# speedup vs baseline: 1.5315x; 1.2071x over previous
import functools

import jax
import jax.numpy as jnp
from jax import lax
from jax.experimental import pallas as pl
from jax.experimental.pallas import tpu as pltpu
from jax.experimental.pallas import tpu_sc as plsc

N = 10000
E = 640000
B = 64
K = 5
K2 = K * K

NTILES = 32
CH = 64
NCHUNK = 10016
CPT = NCHUNK // NTILES
EP = NCHUNK * CH
NP = 10240
RPT = NP // 16


_PCB = 32


def _prep_body(s_ref, d_ref, p0_ref, p1_ref, ei_ref, ew_ref):
    p0 = p0_ref[...]
    p1 = p1_ref[...]
    v0 = jnp.clip(p0, 0.0, 1.0) * (K - 1.0)
    v1 = jnp.clip(p1, 0.0, 1.0) * (K - 1.0)
    i00f = jnp.clip(jnp.floor(v0), 0.0, K - 1.0)
    i10f = jnp.clip(jnp.floor(v1), 0.0, K - 1.0)
    f0 = v0 - i00f
    f1 = v1 - i10f
    i00 = i00f.astype(jnp.int32)
    i01 = jnp.minimum(i00 + 1, K - 1)
    i10 = i10f.astype(jnp.int32)
    i11 = jnp.minimum(i10 + 1, K - 1)
    row = lax.broadcasted_iota(jnp.int32, (_PCB, CH), 0) + pl.program_id(0) * _PCB
    col = lax.broadcasted_iota(jnp.int32, (_PCB, CH), 1)
    valid = ((row * CH + col) < E).astype(jnp.float32)
    base = s_ref[...] * K2
    ei_ref[:, 0, :] = base + i00 * K + i10
    ei_ref[:, 1, :] = base + i00 * K + i11
    ei_ref[:, 2, :] = base + i01 * K + i10
    ei_ref[:, 3, :] = base + i01 * K + i11
    ei_ref[:, 4, :] = d_ref[...]
    ew_ref[:, 0, :] = (1.0 - f0) * (1.0 - f1) * valid
    ew_ref[:, 1, :] = (1.0 - f0) * f1 * valid
    ew_ref[:, 2, :] = f0 * (1.0 - f1) * valid
    ew_ref[:, 3, :] = f0 * f1 * valid


def _prep(src2, dst2, p02, p12):
    return pl.pallas_call(
        _prep_body,
        grid=(NCHUNK // _PCB,),
        in_specs=[pl.BlockSpec((_PCB, CH), lambda i: (i, 0))] * 4,
        out_specs=(pl.BlockSpec((_PCB, 5, CH), lambda i: (i, 0, 0)),
                   pl.BlockSpec((_PCB, 4, CH), lambda i: (i, 0, 0))),
        out_shape=(jax.ShapeDtypeStruct((NCHUNK, 5, CH), jnp.int32),
                   jax.ShapeDtypeStruct((NCHUNK, 4, CH), jnp.float32)),
    )(src2, dst2, p02, p12)


def _table_body(x_ref, w_ref, o_ref):
    o_ref[...] = jnp.dot(x_ref[...], w_ref[...],
                         preferred_element_type=jnp.float32)


def _make_table(x, Wf):
    cin = x.shape[1]
    w = Wf.shape[1]
    Nb = 2000
    return pl.pallas_call(
        _table_body,
        grid=(N // Nb,),
        in_specs=[pl.BlockSpec((Nb, cin), lambda i: (i, 0)),
                  pl.BlockSpec((cin, w), lambda i: (0, 0))],
        out_specs=pl.BlockSpec((Nb, w), lambda i: (i, 0)),
        out_shape=jax.ShapeDtypeStruct((N, w), jnp.float32),
    )(x, Wf)


def _sc_agg(table2d, eidx, ew, cp):
    grp = cp // 16
    mesh = plsc.VectorSubcoreMesh(core_axis_name="c", subcore_axis_name="s")

    @functools.partial(
        pl.kernel,
        out_type=jax.ShapeDtypeStruct((2, NP, cp), jnp.float32),
        mesh=mesh,
        compiler_params=pltpu.CompilerParams(use_tc_tiling_on_sc=False),
        scratch_types=[
            pltpu.VMEM((2, 5, CH), jnp.int32),
            pltpu.VMEM((2, 4, CH), jnp.float32),
            pltpu.VMEM((4, CH, cp), jnp.float32),
            pltpu.VMEM((CH, cp), jnp.float32),
            pltpu.VMEM_SHARED((NP, cp), jnp.float32),
            pltpu.SemaphoreType.DMA,
            pltpu.SemaphoreType.DMA,
        ],
    )
    def sc_kernel(table_hbm, eidx_hbm, ew_hbm, out_hbm,
                  exv, ewv, rb, yb, acc, gsem, ssem):
        c = lax.axis_index("c")
        s = lax.axis_index("s")
        gw = c * 16 + s

        def zrow(i, carry):
            for j in range(grp):
                yb[i, pl.ds(j * 16, 16)] = jnp.zeros((16,), jnp.float32)
            return carry
        lax.fori_loop(0, CH, zrow, 0)
        base = s * RPT
        for i in range(RPT // CH):
            pltpu.sync_copy(yb, acc.at[pl.ds(base + i * CH, CH)])
        plsc.subcore_barrier()

        def issue_gathers(pb):
            for j in range(4):
                pltpu.async_copy(table_hbm.at[exv.at[pb, j]], rb.at[j], gsem)

        def wait_gathers(pb):
            for j in range(4):
                pltpu.make_async_copy(table_hbm.at[exv.at[pb, j]], rb.at[j],
                                      gsem).wait()

        def issue_stage(t, pb):
            ci = gw * CPT + t
            pltpu.async_copy(eidx_hbm.at[ci], exv.at[pb], ssem)
            pltpu.async_copy(ew_hbm.at[ci], ewv.at[pb], ssem)

        def wait_stage(t, pb):
            ci = gw * CPT + t
            pltpu.make_async_copy(eidx_hbm.at[ci], exv.at[pb], ssem).wait()
            pltpu.make_async_copy(ew_hbm.at[ci], ewv.at[pb], ssem).wait()

        def chunk_body(t, p):
            @pl.when(p == 0)
            def _g0():
                issue_gathers(0)

            @pl.when(p == 1)
            def _g1():
                issue_gathers(1)

            @pl.when((t + 1 < CPT) & (p == 0))
            def _s1():
                issue_stage(t + 1, 1)

            @pl.when((t + 1 < CPT) & (p == 1))
            def _s0():
                issue_stage(t + 1, 0)

            @pl.when(p == 0)
            def _w0():
                wait_gathers(0)

            @pl.when(p == 1)
            def _w1():
                wait_gathers(1)

            def ebody(g, ecarry):
                gsl = pl.ds(g * 16, 16)
                wv0 = ewv[p, 0, gsl]
                wv1 = ewv[p, 1, gsl]
                wv2 = ewv[p, 2, gsl]
                wv3 = ewv[p, 3, gsl]
                for l in range(16):
                    e = g * 16 + l
                    w0 = wv0[l]
                    w1 = wv1[l]
                    w2 = wv2[l]
                    w3 = wv3[l]
                    for j in range(grp):
                        sl = pl.ds(j * 16, 16)
                        yb[e, sl] = (w0 * rb[0, e, sl] + w1 * rb[1, e, sl]
                                     + w2 * rb[2, e, sl] + w3 * rb[3, e, sl])
                return ecarry
            lax.fori_loop(0, CH // 16, ebody, 0)

            @pl.when(p == 0)
            def _a0():
                pltpu.sync_copy(yb, acc.at[exv.at[0, 4]], add=True)

            @pl.when(p == 1)
            def _a1():
                pltpu.sync_copy(yb, acc.at[exv.at[1, 4]], add=True)

            @pl.when((t + 1 < CPT) & (p == 0))
            def _ws1():
                wait_stage(t + 1, 1)

            @pl.when((t + 1 < CPT) & (p == 1))
            def _ws0():
                wait_stage(t + 1, 0)

            return 1 - p
        pltpu.sync_copy(eidx_hbm.at[gw * CPT], exv.at[0])
        pltpu.sync_copy(ew_hbm.at[gw * CPT], ewv.at[0])
        lax.fori_loop(0, CPT, chunk_body, jnp.int32(0))

        plsc.subcore_barrier()
        pltpu.sync_copy(acc.at[pl.ds(base, RPT)],
                        out_hbm.at[c, pl.ds(base, RPT)])

    return sc_kernel(table2d, eidx, ew)


def _elu(h):
    return jnp.where(h > 0, h, jnp.exp(h) - 1.0)


def _finish1_body(p_ref, x_ref, r_ref, b_ref, h_ref, d_ref):
    a = p_ref[0] + p_ref[1]
    deg = jnp.maximum(a[:, 32:33], 1.0)
    h = a[:, :32] / deg + jnp.dot(x_ref[...], r_ref[...],
                                  preferred_element_type=jnp.float32) + b_ref[...]
    h_ref[...] = _elu(h)
    d_ref[...] = deg


def _finish1(part, x, root, bias):
    return pl.pallas_call(
        _finish1_body,
        out_shape=(jax.ShapeDtypeStruct((N, 32), jnp.float32),
                   jax.ShapeDtypeStruct((N, 1), jnp.float32)),
    )(part, x, root, bias)


def _finish_body(cout, p_ref, x_ref, r_ref, b_ref, d_ref, h_ref):
    a = p_ref[0] + p_ref[1]
    agg = a[:, :cout] if cout != a.shape[1] else a
    h = agg / d_ref[...] + jnp.dot(x_ref[...], r_ref[...],
                                   preferred_element_type=jnp.float32) + b_ref[...]
    h_ref[...] = _elu(h)


def _finish(part, x, root, bias, deg, cout):
    return pl.pallas_call(
        functools.partial(_finish_body, cout),
        out_shape=jax.ShapeDtypeStruct((N, cout), jnp.float32),
    )(part, x, root, bias, deg)


def _pool_fc_body(sl_ref, h_ref, wfc_ref, bfc_ref, out_ref):
    n_iota1 = lax.broadcasted_iota(jnp.int32, (1, N), 1)
    seg = jnp.zeros((1, N), jnp.int32)
    for j in range(1, B):
        seg = seg + jnp.where(sl_ref[j] <= n_iota1, 1, 0)
    b_iota = lax.broadcasted_iota(jnp.int32, (B, N), 0)
    M = (seg == b_iota).astype(jnp.float32)
    cnt = jnp.sum(M, axis=1, keepdims=True)
    sums = jnp.dot(M, h_ref[...], preferred_element_type=jnp.float32)
    g = sums / jnp.maximum(cnt, 1.0)
    logits = jnp.dot(g, wfc_ref[...], preferred_element_type=jnp.float32) + bfc_ref[...]
    m = jnp.max(logits, axis=1, keepdims=True)
    z = logits - m
    out_ref[...] = z - jnp.log(jnp.sum(jnp.exp(z), axis=1, keepdims=True))


def _pool_fc(h, sl, Wfc, bfc):
    return pl.pallas_call(
        _pool_fc_body,
        out_shape=jax.ShapeDtypeStruct((B, Wfc.shape[1]), jnp.float32),
        in_specs=[
            pl.BlockSpec(memory_space=pltpu.SMEM),
            pl.BlockSpec(memory_space=pltpu.VMEM),
            pl.BlockSpec(memory_space=pltpu.VMEM),
            pl.BlockSpec(memory_space=pltpu.VMEM),
        ],
        out_specs=pl.BlockSpec(memory_space=pltpu.VMEM),
    )(sl, h, Wfc, bfc)


def kernel(x, edge_index, pseudo, slice_idx, W1, root1, b1, W2, root2, b2,
           W3, root3, b3, Wfc, bfc):
    pad = EP - E
    src2 = jnp.pad(edge_index[0], (0, pad)).reshape(NCHUNK, CH)
    dst2 = jnp.pad(edge_index[1], (0, pad)).reshape(NCHUNK, CH)
    p02 = jnp.pad(pseudo[:, 0], (0, pad)).reshape(NCHUNK, CH)
    p12 = jnp.pad(pseudo[:, 1], (0, pad)).reshape(NCHUNK, CH)
    eidx, ew = _prep(src2, dst2, p02, p12)

    x1 = jnp.concatenate([x, jnp.ones((N, 1), x.dtype)], axis=1)
    W1p = (jnp.zeros((K2, 9, 48), jnp.float32)
           .at[:, :8, :32].set(W1)
           .at[:, 8, 32].set(1.0))
    t1 = _make_table(x1, W1p.transpose(1, 0, 2).reshape(9, K2 * 48))
    part1 = _sc_agg(t1.reshape(N * K2, 48), eidx, ew, 48)
    h1, deg = _finish1(part1[:, :N, :], x, root1, b1)

    t2 = _make_table(h1, W2.transpose(1, 0, 2).reshape(32, K2 * 64))
    part2 = _sc_agg(t2.reshape(N * K2, 64), eidx, ew, 64)
    h2 = _finish(part2[:, :N, :], h1, root2, b2, deg, 64)

    W3p = jnp.pad(W3, ((0, 0), (0, 0), (0, 4)))
    t3 = _make_table(h2, W3p.transpose(1, 0, 2).reshape(64, K2 * 128))
    part3 = _sc_agg(t3.reshape(N * K2, 128), eidx, ew, 128)
    h3 = _finish(part3[:, :N, :], h2, root3, b3, deg, 124)

    return _pool_fc(h3, slice_idx, Wfc, bfc)
